# baseline (speedup 1.0000x reference)
import jax
import jax.numpy as jnp
from jax.experimental import pallas as pl
import jax.experimental.pallas.tpu as pltpu

_WAY = 5
_B = 8
_D = 3
_K = 4
_BK = _B // _K


def _body(x_hbm, s_hbm, o_hbm, xb, sb, yb, in_sem, s_sem, out_sem):
    i = pl.program_id(0)
    S = pl.num_programs(0)
    G = x_hbm.shape[0] // _WAY
    bpw = G // _B
    slot = jax.lax.rem(i, 2)
    nslot = jax.lax.rem(i + 1, 2)
    yslot = jax.lax.rem(i, _D)

    def start_in(step, sl):
        for k in range(_K):
            pltpu.make_async_copy(
                x_hbm.at[pl.ds(step * _B + k * _BK, _BK)],
                xb.at[sl, pl.ds(k * _BK, _BK)], in_sem.at[sl, k]).start()
        pltpu.make_async_copy(s_hbm.at[pl.ds(step * _B, _B)],
                              sb.at[sl], s_sem.at[sl]).start()

    @pl.when(i == 0)
    def _():
        start_in(i, slot)

    @pl.when(i + 1 < S)
    def _():
        start_in(i + 1, nslot)

    for k in range(_K):
        pltpu.make_async_copy(
            x_hbm.at[pl.ds(i * _B + k * _BK, _BK)],
            xb.at[slot, pl.ds(k * _BK, _BK)], in_sem.at[slot, k]).wait()
    pltpu.make_async_copy(s_hbm.at[pl.ds(i * _B, _B)],
                          sb.at[slot], s_sem.at[slot]).wait()

    @pl.when(i >= _D)
    def _():
        for j in range(2):
            for k in range(_K):
                pltpu.make_async_copy(
                    yb.at[yslot, pl.ds(k * _BK, _BK)],
                    o_hbm.at[0, j, pl.ds(k * _BK, _BK)],
                    out_sem.at[yslot, j, k]).wait()

    st = jnp.swapaxes(sb[slot], 0, 1)
    for b in range(_B):
        yb[yslot, b] = xb[slot, b] * st[:, b][:, None]

    way = i // bpw
    g0 = jax.lax.rem(i, bpw) * _B
    for j in range(2):
        for k in range(_K):
            pltpu.make_async_copy(
                yb.at[yslot, pl.ds(k * _BK, _BK)],
                o_hbm.at[way, j, pl.ds(g0 + k * _BK, _BK)],
                out_sem.at[yslot, j, k]).start()

    @pl.when(i == S - 1)
    def _():
        for kk in range(_D):
            for j in range(2):
                for k in range(_K):
                    pltpu.make_async_copy(
                        yb.at[kk, pl.ds(k * _BK, _BK)],
                        o_hbm.at[0, j, pl.ds(k * _BK, _BK)],
                        out_sem.at[kk, j, k]).wait()


def kernel(x, s_ca, shuffle_num):
    N, c, h, w = x.shape
    hw = h * w
    G = N // _WAY

    x3 = x.reshape(N, c, hw)
    s2 = s_ca.reshape(N, c)

    out = pl.pallas_call(
        _body,
        grid=(N // _B,),
        in_specs=[
            pl.BlockSpec(memory_space=pl.ANY),
            pl.BlockSpec(memory_space=pl.ANY),
        ],
        out_specs=pl.BlockSpec(memory_space=pl.ANY),
        out_shape=jax.ShapeDtypeStruct((_WAY, 2, G, c, hw), x.dtype),
        scratch_shapes=[
            pltpu.VMEM((2, _B, c, hw), x.dtype),
            pltpu.VMEM((2, _B, c), x.dtype),
            pltpu.VMEM((_D, _B, c, hw), x.dtype),
            pltpu.SemaphoreType.DMA((2, _K)),
            pltpu.SemaphoreType.DMA((2,)),
            pltpu.SemaphoreType.DMA((_D, 2, _K)),
        ],
    )(x3, s2)
    return out.reshape(2 * N, c, h, w)
